# Initial kernel scaffold; baseline (speedup 1.0000x reference)
import functools

import jax
import jax.numpy as jnp
from jax import lax
from jax.experimental import pallas as pl
from jax.experimental.pallas import tpu as pltpu
from jax.experimental.pallas import tpu_sc as plsc

_M = 1000000
_MP = 1 << 20
_B = 16384
_L = 200
_N = _B * _L
_LANES = 128
_ROWS = _N // _LANES
_NC = 2
_NS = 16
_NW = _NC * _NS
_ROWS_PER_W = _ROWS // _NW
_CHUNK_ROWS = 32
_NCHUNK = _ROWS_PER_W // _CHUNK_ROWS
_ACC_PER_TILE = _MP // _NS
_ZBUF = 4096


def _sc_scatter_partials(idx2d, w2d):
    mesh = plsc.VectorSubcoreMesh(core_axis_name="c", subcore_axis_name="s")

    @functools.partial(
        pl.kernel,
        mesh=mesh,
        out_type=jax.ShapeDtypeStruct((_NC, _MP), jnp.float32),
        scratch_types=[
            pltpu.VMEM_SHARED((_MP,), jnp.float32),
            pltpu.VMEM((_ZBUF,), jnp.float32),
            pltpu.VMEM((_CHUNK_ROWS, _LANES), jnp.int32),
            pltpu.VMEM((_CHUNK_ROWS, _LANES), jnp.int32),
            pltpu.VMEM((_CHUNK_ROWS, _LANES), jnp.float32),
            pltpu.VMEM((_CHUNK_ROWS, _LANES), jnp.float32),
            pltpu.SemaphoreType.DMA,
            pltpu.SemaphoreType.DMA,
            pltpu.SemaphoreType.DMA,
        ],
    )
    def scatter_kernel(idx_hbm, w_hbm, out_hbm, acc, zbuf, ib0, ib1, wb0,
                       wb1, sld0, sld1, ssc):
        cid = lax.axis_index("c")
        sid = lax.axis_index("s")
        wid = sid * _NC + cid

        def _zero_body(i, _):
            zbuf[pl.ds(i * 16, 16)] = jnp.zeros((16,), jnp.float32)
            return 0

        lax.fori_loop(0, _ZBUF // 16, _zero_body, 0)
        acc_base = sid * _ACC_PER_TILE
        for k in range(_ACC_PER_TILE // _ZBUF):
            pltpu.sync_copy(zbuf, acc.at[pl.ds(acc_base + k * _ZBUF, _ZBUF)])
        plsc.subcore_barrier()

        ibufs, wbufs, slds = (ib0, ib1), (wb0, wb1), (sld0, sld1)
        row0 = wid * _ROWS_PER_W

        def _start_load(c, par):
            r = row0 + c * _CHUNK_ROWS
            d1 = pltpu.async_copy(
                idx_hbm.at[pl.ds(r, _CHUNK_ROWS)], ibufs[par], slds[par])
            d2 = pltpu.async_copy(
                w_hbm.at[pl.ds(r, _CHUNK_ROWS)], wbufs[par], slds[par])
            return d1, d2

        pending = _start_load(0, 0)
        for c in range(_NCHUNK):
            par = c % 2
            for d in pending:
                d.wait()
            if c + 1 < _NCHUNK:
                pending = _start_load(c + 1, (c + 1) % 2)
            pltpu.async_copy(
                wbufs[par], acc.at[ibufs[par]], ssc, add=True).wait()

        plsc.subcore_barrier()
        out_base = sid * _ACC_PER_TILE
        pltpu.sync_copy(
            acc.at[pl.ds(out_base, _ACC_PER_TILE)],
            out_hbm.at[cid, pl.ds(out_base, _ACC_PER_TILE)])

    return scatter_kernel(idx2d, w2d)


def _tc_combine(partials):
    def _add_body(p_ref, o_ref):
        o_ref[...] = p_ref[0] + p_ref[1]

    return pl.pallas_call(
        _add_body,
        grid=(16,),
        in_specs=[pl.BlockSpec((2, 512, 128), lambda i: (0, i, 0))],
        out_specs=pl.BlockSpec((512, 128), lambda i: (i, 0)),
        out_shape=jax.ShapeDtypeStruct((_MP // 128, 128), jnp.float32),
    )(partials.reshape(2, _MP // 128, 128))


def kernel(x, w):
    idx2d = x.reshape(_ROWS, _LANES)
    w2d = w.reshape(_ROWS, _LANES)
    partials = _sc_scatter_partials(idx2d, w2d)
    return _tc_combine(partials).reshape(-1)[:_M]


# speedup vs baseline: 36.0772x; 36.0772x over previous
import functools

import jax
import jax.numpy as jnp
from jax import lax
from jax.experimental import pallas as pl
from jax.experimental.pallas import tpu as pltpu
from jax.experimental.pallas import tpu_sc as plsc

_M = 1000000
_MP = 1 << 20
_B = 16384
_L = 200
_N = _B * _L
_NC = 2
_NS = 16
_NW = _NC * _NS
_PAIRS_PER_W = _N // _NW
_CHUNK = 4096
_NCHUNK = _PAIRS_PER_W // _CHUNK
_ACC_PER_TILE = _MP // _NS
_ZBUF = 4096


def _sc_scatter_partials(idx2d, w2d):
    mesh = plsc.VectorSubcoreMesh(core_axis_name="c", subcore_axis_name="s")

    @functools.partial(
        pl.kernel,
        mesh=mesh,
        out_type=jax.ShapeDtypeStruct((_NC, _MP), jnp.float32),
        scratch_types=[
            pltpu.VMEM_SHARED((_MP,), jnp.float32),
            pltpu.VMEM((_ZBUF,), jnp.float32),
            pltpu.VMEM((_CHUNK,), jnp.int32),
            pltpu.VMEM((_CHUNK,), jnp.int32),
            pltpu.VMEM((_CHUNK,), jnp.float32),
            pltpu.VMEM((_CHUNK,), jnp.float32),
            pltpu.SemaphoreType.DMA,
            pltpu.SemaphoreType.DMA,
            pltpu.SemaphoreType.DMA,
        ],
    )
    def scatter_kernel(idx_hbm, w_hbm, out_hbm, acc, zbuf, ib0, ib1, wb0,
                       wb1, sld0, sld1, ssc):
        cid = lax.axis_index("c")
        sid = lax.axis_index("s")
        wid = sid * _NC + cid

        def _zero_body(i, _):
            zbuf[pl.ds(i * 16, 16)] = jnp.zeros((16,), jnp.float32)
            return 0

        lax.fori_loop(0, _ZBUF // 16, _zero_body, 0)
        acc_base = sid * _ACC_PER_TILE
        for k in range(_ACC_PER_TILE // _ZBUF):
            pltpu.sync_copy(zbuf, acc.at[pl.ds(acc_base + k * _ZBUF, _ZBUF)])
        plsc.subcore_barrier()

        ibufs, wbufs, slds = (ib0, ib1), (wb0, wb1), (sld0, sld1)
        base0 = wid * _PAIRS_PER_W

        def _start_load(c, par):
            r = base0 + c * _CHUNK
            d1 = pltpu.async_copy(
                idx_hbm.at[pl.ds(r, _CHUNK)], ibufs[par], slds[par])
            d2 = pltpu.async_copy(
                w_hbm.at[pl.ds(r, _CHUNK)], wbufs[par], slds[par])
            return d1, d2

        pending = _start_load(0, 0)
        for c in range(_NCHUNK):
            par = c % 2
            for d in pending:
                d.wait()
            if c + 1 < _NCHUNK:
                pending = _start_load(c + 1, (c + 1) % 2)
            pltpu.async_copy(
                wbufs[par], acc.at[ibufs[par]], ssc, add=True).wait()

        plsc.subcore_barrier()
        out_base = sid * _ACC_PER_TILE
        pltpu.sync_copy(
            acc.at[pl.ds(out_base, _ACC_PER_TILE)],
            out_hbm.at[cid, pl.ds(out_base, _ACC_PER_TILE)])

    return scatter_kernel(idx2d, w2d)


def _tc_combine(partials):
    def _add_body(p_ref, o_ref):
        o_ref[...] = p_ref[0] + p_ref[1]

    return pl.pallas_call(
        _add_body,
        grid=(16,),
        in_specs=[pl.BlockSpec((2, 512, 128), lambda i: (0, i, 0))],
        out_specs=pl.BlockSpec((512, 128), lambda i: (i, 0)),
        out_shape=jax.ShapeDtypeStruct((_MP // 128, 128), jnp.float32),
    )(partials.reshape(2, _MP // 128, 128))


def kernel(x, w):
    partials = _sc_scatter_partials(x.reshape(_N), w.reshape(_N))
    return _tc_combine(partials).reshape(-1)[:_M]


# speedup vs baseline: 38.9146x; 1.0786x over previous
import functools

import jax
import jax.numpy as jnp
from jax import lax
from jax.experimental import pallas as pl
from jax.experimental.pallas import tpu as pltpu
from jax.experimental.pallas import tpu_sc as plsc

_M = 1000000
_MP = 1 << 20
_B = 16384
_L = 200
_N = _B * _L
_NC = 2
_NS = 16
_NW = _NC * _NS
_PAIRS_PER_W = _N // _NW
_CHUNK = 4096
_NCHUNK = _PAIRS_PER_W // _CHUNK
_ACC_PER_TILE = _MP // _NS
_ZBUF = 4096

_CW = 31248
_TAIL = _M - _NW * _CW


def _sc_scatter_partials(idx_hbm_arr, w_hbm_arr):
    mesh = plsc.VectorSubcoreMesh(core_axis_name="c", subcore_axis_name="s")

    @functools.partial(
        pl.kernel,
        mesh=mesh,
        out_type=(jax.ShapeDtypeStruct((_MP,), jnp.float32),
                  jax.ShapeDtypeStruct((_MP,), jnp.float32)),
        scratch_types=[
            pltpu.VMEM_SHARED((_MP,), jnp.float32),
            pltpu.VMEM((_ZBUF,), jnp.float32),
            [pltpu.VMEM((_CHUNK,), jnp.int32) for _ in range(4)],
            [pltpu.VMEM((_CHUNK,), jnp.float32) for _ in range(4)],
            [pltpu.SemaphoreType.DMA for _ in range(4)],
            [pltpu.SemaphoreType.DMA for _ in range(4)],
            pltpu.SemaphoreType.DMA,
        ],
    )
    def scatter_kernel(idx_hbm, w_hbm, p0_hbm, p1_hbm, acc, zbuf, ibufs,
                       wbufs, slds, sscs, szb):
        cid = lax.axis_index("c")
        sid = lax.axis_index("s")
        wid = sid * _NC + cid
        base0 = wid * _PAIRS_PER_W

        def _start_load(c):
            b = c % 4
            r = base0 + c * _CHUNK
            d1 = pltpu.async_copy(
                idx_hbm.at[pl.ds(r, _CHUNK)], ibufs[b], slds[b])
            d2 = pltpu.async_copy(
                w_hbm.at[pl.ds(r, _CHUNK)], wbufs[b], slds[b])
            return d1, d2

        lds = [_start_load(0), _start_load(1), None, None]

        def _zero_body(i, _):
            zbuf[pl.ds(i * 16, 16)] = jnp.zeros((16,), jnp.float32)
            return 0

        lax.fori_loop(0, _ZBUF // 16, _zero_body, 0)
        acc_base = sid * _ACC_PER_TILE
        zds = [
            pltpu.async_copy(
                zbuf, acc.at[pl.ds(acc_base + k * _ZBUF, _ZBUF)], szb)
            for k in range(_ACC_PER_TILE // _ZBUF)
        ]
        for d in zds:
            d.wait()
        plsc.subcore_barrier()

        scats = [None, None, None, None]
        for c in range(_NCHUNK):
            b = c % 4
            for d in lds[b]:
                d.wait()
            scats[b] = pltpu.async_copy(
                wbufs[b], acc.at[ibufs[b]], sscs[b], add=True)
            if c >= 2:
                scats[(c - 2) % 4].wait()
                scats[(c - 2) % 4] = None
            if c + 2 < _NCHUNK:
                lds[(c + 2) % 4] = _start_load(c + 2)
        for s in scats:
            if s is not None:
                s.wait()

        plsc.subcore_barrier()
        sl = pl.ds(acc_base, _ACC_PER_TILE)

        @pl.when(cid == 0)
        def _():
            pltpu.sync_copy(acc.at[sl], p0_hbm.at[sl])

        @pl.when(cid == 1)
        def _():
            pltpu.sync_copy(acc.at[sl], p1_hbm.at[sl])

    return scatter_kernel(idx_hbm_arr, w_hbm_arr)


def _sc_combine(p0_arr, p1_arr):
    mesh = plsc.VectorSubcoreMesh(core_axis_name="c", subcore_axis_name="s")

    @functools.partial(
        pl.kernel,
        mesh=mesh,
        out_type=jax.ShapeDtypeStruct((_M,), jnp.float32),
        scratch_types=[
            pltpu.VMEM((_CW,), jnp.float32),
            pltpu.VMEM((_CW,), jnp.float32),
            pltpu.VMEM((16,), jnp.float32),
            pltpu.VMEM((16,), jnp.float32),
            pltpu.SemaphoreType.DMA,
        ],
    )
    def combine_kernel(p0_hbm, p1_hbm, out_hbm, b0, b1, t0, t1, sem):
        cid = lax.axis_index("c")
        sid = lax.axis_index("s")
        wid = sid * _NC + cid
        off = wid * _CW
        d0 = pltpu.async_copy(p0_hbm.at[pl.ds(off, _CW)], b0, sem)
        d1 = pltpu.async_copy(p1_hbm.at[pl.ds(off, _CW)], b1, sem)
        d0.wait()
        d1.wait()

        def _add_body(i, _):
            s = pl.ds(i * 16, 16)
            b0[s] = b0[s] + b1[s]
            return 0

        lax.fori_loop(0, _CW // 16, _add_body, 0)
        pltpu.sync_copy(b0, out_hbm.at[pl.ds(off, _CW)])

        @pl.when(wid == 0)
        def _():
            base = _NW * _CW
            for k in range(_TAIL // 16):
                s = pl.ds(base + k * 16, 16)
                pltpu.sync_copy(p0_hbm.at[s], t0)
                pltpu.sync_copy(p1_hbm.at[s], t1)
                t0[pl.ds(0, 16)] = t0[pl.ds(0, 16)] + t1[pl.ds(0, 16)]
                pltpu.sync_copy(t0, out_hbm.at[s])

    return combine_kernel(p0_arr, p1_arr)


def kernel(x, w):
    p0, p1 = _sc_scatter_partials(x.reshape(_N), w.reshape(_N))
    return _sc_combine(p0, p1)


# speedup vs baseline: 41.5103x; 1.0667x over previous
import functools

import jax
import jax.numpy as jnp
from jax import lax
from jax.experimental import pallas as pl
from jax.experimental.pallas import tpu as pltpu
from jax.experimental.pallas import tpu_sc as plsc

_M = 1000000
_MP = 1 << 20
_B = 16384
_L = 200
_N = _B * _L
_NC = 2
_NS = 16
_NW = _NC * _NS
_PAIRS_PER_W = _N // _NW
_CHUNK = 4096
_NCHUNK = _PAIRS_PER_W // _CHUNK
_ACC_PER_TILE = _MP // _NS
_ZBUF = 4096

_CW = 31232
_TAIL = _M - _NW * _CW


def _sc_scatter_partials(idx_hbm_arr, w_hbm_arr):
    mesh = plsc.VectorSubcoreMesh(core_axis_name="c", subcore_axis_name="s")

    @functools.partial(
        pl.kernel,
        mesh=mesh,
        out_type=(jax.ShapeDtypeStruct((_MP,), jnp.float32),
                  jax.ShapeDtypeStruct((_MP,), jnp.float32)),
        scratch_types=[
            pltpu.VMEM_SHARED((_MP,), jnp.float32),
            pltpu.VMEM((_ZBUF,), jnp.float32),
            [pltpu.VMEM((_CHUNK,), jnp.int32) for _ in range(4)],
            [pltpu.VMEM((_CHUNK,), jnp.float32) for _ in range(4)],
            [pltpu.SemaphoreType.DMA for _ in range(4)],
            [pltpu.SemaphoreType.DMA for _ in range(4)],
            pltpu.SemaphoreType.DMA,
        ],
    )
    def scatter_kernel(idx_hbm, w_hbm, p0_hbm, p1_hbm, acc, zbuf, ibufs,
                       wbufs, slds, sscs, szb):
        cid = lax.axis_index("c")
        sid = lax.axis_index("s")
        wid = sid * _NC + cid
        base0 = wid * _PAIRS_PER_W

        def _start_load(c):
            b = c % 4
            r = base0 + c * _CHUNK
            d1 = pltpu.async_copy(
                idx_hbm.at[pl.ds(r, _CHUNK)], ibufs[b], slds[b])
            d2 = pltpu.async_copy(
                w_hbm.at[pl.ds(r, _CHUNK)], wbufs[b], slds[b])
            return d1, d2

        lds = [_start_load(0), _start_load(1), None, None]

        def _zero_body(i, _):
            zbuf[pl.ds(i * 16, 16)] = jnp.zeros((16,), jnp.float32)
            return 0

        lax.fori_loop(0, _ZBUF // 16, _zero_body, 0)
        acc_base = sid * _ACC_PER_TILE
        zds = [
            pltpu.async_copy(
                zbuf, acc.at[pl.ds(acc_base + k * _ZBUF, _ZBUF)], szb)
            for k in range(_ACC_PER_TILE // _ZBUF)
        ]
        for d in zds:
            d.wait()
        plsc.subcore_barrier()

        scats = [None, None, None, None]
        for c in range(_NCHUNK):
            b = c % 4
            for d in lds[b]:
                d.wait()
            scats[b] = pltpu.async_copy(
                wbufs[b], acc.at[ibufs[b]], sscs[b], add=True)
            if c >= 2:
                scats[(c - 2) % 4].wait()
                scats[(c - 2) % 4] = None
            if c + 2 < _NCHUNK:
                lds[(c + 2) % 4] = _start_load(c + 2)
        for s in scats:
            if s is not None:
                s.wait()

        plsc.subcore_barrier()
        sl = pl.ds(acc_base, _ACC_PER_TILE)

        @pl.when(cid == 0)
        def _():
            pltpu.sync_copy(acc.at[sl], p0_hbm.at[sl])

        @pl.when(cid == 1)
        def _():
            pltpu.sync_copy(acc.at[sl], p1_hbm.at[sl])

    return scatter_kernel(idx_hbm_arr, w_hbm_arr)


def _sc_combine(p0_arr, p1_arr):
    mesh = plsc.VectorSubcoreMesh(core_axis_name="c", subcore_axis_name="s")

    @functools.partial(
        pl.kernel,
        mesh=mesh,
        out_type=jax.ShapeDtypeStruct((_M,), jnp.float32),
        scratch_types=[
            pltpu.VMEM((_CW,), jnp.float32),
            pltpu.VMEM((_CW,), jnp.float32),
            pltpu.VMEM((_TAIL,), jnp.float32),
            pltpu.VMEM((_TAIL,), jnp.float32),
            pltpu.SemaphoreType.DMA,
        ],
    )
    def combine_kernel(p0_hbm, p1_hbm, out_hbm, b0, b1, t0, t1, sem):
        cid = lax.axis_index("c")
        sid = lax.axis_index("s")
        wid = sid * _NC + cid
        off = wid * _CW
        d0 = pltpu.async_copy(p0_hbm.at[pl.ds(off, _CW)], b0, sem)
        d1 = pltpu.async_copy(p1_hbm.at[pl.ds(off, _CW)], b1, sem)
        d0.wait()
        d1.wait()

        @plsc.parallel_loop(0, _CW // 16, 1, unroll=8)
        def _add(i):
            plsc.addupdate(b0.at[pl.ds(i * 16, 16)], b1[pl.ds(i * 16, 16)])

        pltpu.sync_copy(b0, out_hbm.at[pl.ds(off, _CW)])

        @pl.when(wid == 0)
        def _():
            base = _NW * _CW
            s = pl.ds(base, _TAIL)
            pltpu.sync_copy(p0_hbm.at[s], t0)
            pltpu.sync_copy(p1_hbm.at[s], t1)

            @plsc.parallel_loop(0, _TAIL // 16, 1, unroll=4)
            def _add_tail(k):
                plsc.addupdate(
                    t0.at[pl.ds(k * 16, 16)], t1[pl.ds(k * 16, 16)])

            pltpu.sync_copy(t0, out_hbm.at[s])

    return combine_kernel(p0_arr, p1_arr)


def kernel(x, w):
    p0, p1 = _sc_scatter_partials(x.reshape(_N), w.reshape(_N))
    return _sc_combine(p0, p1)


# speedup vs baseline: 41.7047x; 1.0047x over previous
import functools

import jax
import jax.numpy as jnp
from jax import lax
from jax.experimental import pallas as pl
from jax.experimental.pallas import tpu as pltpu
from jax.experimental.pallas import tpu_sc as plsc

_M = 1000000
_MP = 1 << 20
_B = 16384
_L = 200
_N = _B * _L
_NC = 2
_NS = 16
_NW = _NC * _NS
_PAIRS_PER_W = _N // _NW
_CHUNK = 6400
_NCHUNK = _PAIRS_PER_W // _CHUNK
_ACC_PER_TILE = _MP // _NS
_ZBUF = 4096

_CW = 31232
_TAIL = _M - _NW * _CW


def _sc_scatter_partials(idx_hbm_arr, w_hbm_arr):
    mesh = plsc.VectorSubcoreMesh(core_axis_name="c", subcore_axis_name="s")

    @functools.partial(
        pl.kernel,
        mesh=mesh,
        out_type=(jax.ShapeDtypeStruct((_MP,), jnp.float32),
                  jax.ShapeDtypeStruct((_MP,), jnp.float32)),
        scratch_types=[
            pltpu.VMEM_SHARED((_MP,), jnp.float32),
            pltpu.VMEM((_ZBUF,), jnp.float32),
            [pltpu.VMEM((_CHUNK,), jnp.int32) for _ in range(4)],
            [pltpu.VMEM((_CHUNK,), jnp.float32) for _ in range(4)],
            [pltpu.SemaphoreType.DMA for _ in range(4)],
            [pltpu.SemaphoreType.DMA for _ in range(4)],
            pltpu.SemaphoreType.DMA,
        ],
    )
    def scatter_kernel(idx_hbm, w_hbm, p0_hbm, p1_hbm, acc, zbuf, ibufs,
                       wbufs, slds, sscs, szb):
        cid = lax.axis_index("c")
        sid = lax.axis_index("s")
        wid = sid * _NC + cid
        base0 = wid * _PAIRS_PER_W

        def _start_load(c):
            b = c % 4
            r = base0 + c * _CHUNK
            d1 = pltpu.async_copy(
                idx_hbm.at[pl.ds(r, _CHUNK)], ibufs[b], slds[b])
            d2 = pltpu.async_copy(
                w_hbm.at[pl.ds(r, _CHUNK)], wbufs[b], slds[b])
            return d1, d2

        lds = [_start_load(0), _start_load(1), None, None]

        def _zero_body(i, _):
            zbuf[pl.ds(i * 16, 16)] = jnp.zeros((16,), jnp.float32)
            return 0

        lax.fori_loop(0, _ZBUF // 16, _zero_body, 0)
        acc_base = sid * _ACC_PER_TILE
        zds = [
            pltpu.async_copy(
                zbuf, acc.at[pl.ds(acc_base + k * _ZBUF, _ZBUF)], szb)
            for k in range(_ACC_PER_TILE // _ZBUF)
        ]
        for d in zds:
            d.wait()
        plsc.subcore_barrier()

        scats = [None, None, None, None]
        for c in range(_NCHUNK):
            b = c % 4
            for d in lds[b]:
                d.wait()
            scats[b] = pltpu.async_copy(
                wbufs[b], acc.at[ibufs[b]], sscs[b], add=True)
            if c >= 2:
                scats[(c - 2) % 4].wait()
                scats[(c - 2) % 4] = None
            if c + 2 < _NCHUNK:
                lds[(c + 2) % 4] = _start_load(c + 2)
        for s in scats:
            if s is not None:
                s.wait()

        plsc.subcore_barrier()
        sl = pl.ds(acc_base, _ACC_PER_TILE)

        @pl.when(cid == 0)
        def _():
            pltpu.sync_copy(acc.at[sl], p0_hbm.at[sl])

        @pl.when(cid == 1)
        def _():
            pltpu.sync_copy(acc.at[sl], p1_hbm.at[sl])

    return scatter_kernel(idx_hbm_arr, w_hbm_arr)


def _sc_combine(p0_arr, p1_arr):
    mesh = plsc.VectorSubcoreMesh(core_axis_name="c", subcore_axis_name="s")

    @functools.partial(
        pl.kernel,
        mesh=mesh,
        out_type=jax.ShapeDtypeStruct((_M,), jnp.float32),
        scratch_types=[
            pltpu.VMEM((_CW,), jnp.float32),
            pltpu.VMEM((_CW,), jnp.float32),
            pltpu.VMEM((_TAIL,), jnp.float32),
            pltpu.VMEM((_TAIL,), jnp.float32),
            pltpu.SemaphoreType.DMA,
        ],
    )
    def combine_kernel(p0_hbm, p1_hbm, out_hbm, b0, b1, t0, t1, sem):
        cid = lax.axis_index("c")
        sid = lax.axis_index("s")
        wid = sid * _NC + cid
        off = wid * _CW
        d0 = pltpu.async_copy(p0_hbm.at[pl.ds(off, _CW)], b0, sem)
        d1 = pltpu.async_copy(p1_hbm.at[pl.ds(off, _CW)], b1, sem)
        d0.wait()
        d1.wait()

        @plsc.parallel_loop(0, _CW // 16, 1, unroll=8)
        def _add(i):
            plsc.addupdate(b0.at[pl.ds(i * 16, 16)], b1[pl.ds(i * 16, 16)])

        pltpu.sync_copy(b0, out_hbm.at[pl.ds(off, _CW)])

        @pl.when(wid == 0)
        def _():
            base = _NW * _CW
            s = pl.ds(base, _TAIL)
            pltpu.sync_copy(p0_hbm.at[s], t0)
            pltpu.sync_copy(p1_hbm.at[s], t1)

            @plsc.parallel_loop(0, _TAIL // 16, 1, unroll=4)
            def _add_tail(k):
                plsc.addupdate(
                    t0.at[pl.ds(k * 16, 16)], t1[pl.ds(k * 16, 16)])

            pltpu.sync_copy(t0, out_hbm.at[s])

    return combine_kernel(p0_arr, p1_arr)


def kernel(x, w):
    p0, p1 = _sc_scatter_partials(x.reshape(_N), w.reshape(_N))
    return _sc_combine(p0, p1)


# speedup vs baseline: 74.8254x; 1.7942x over previous
import functools

import jax
import jax.numpy as jnp
from jax import lax
from jax.experimental import pallas as pl
from jax.experimental.pallas import tpu as pltpu
from jax.experimental.pallas import tpu_sc as plsc

_M = 1000000
_MP = 1 << 20
_B = 16384
_L = 200
_N = _B * _L
_NC = 2
_NS = 16
_NW = _NC * _NS
_PAIRS_PER_W = _N // _NW
_CHUNK = 4096
_NCHUNK = _PAIRS_PER_W // _CHUNK
_STRIP = 512
_ACC_PER_TILE = _MP // _NS
_ZBUF = 4096

_CW = 31232
_TAIL = _M - _NW * _CW


def _sc_scatter_partials(idx_hbm_arr, w_hbm_arr):
    mesh = plsc.VectorSubcoreMesh(core_axis_name="c", subcore_axis_name="s")

    @functools.partial(
        pl.kernel,
        mesh=mesh,
        out_type=(jax.ShapeDtypeStruct((_MP,), jnp.float32),
                  jax.ShapeDtypeStruct((_MP,), jnp.float32)),
        scratch_types=[
            pltpu.VMEM_SHARED((_MP,), jnp.float32),
            pltpu.VMEM((_ZBUF,), jnp.float32),
            [pltpu.VMEM((_CHUNK,), jnp.int32) for _ in range(4)],
            [pltpu.VMEM((_CHUNK,), jnp.int32) for _ in range(4)],
            [pltpu.VMEM((_CHUNK,), jnp.float32) for _ in range(4)],
            [pltpu.SemaphoreType.DMA for _ in range(4)],
            [pltpu.SemaphoreType.DMA for _ in range(4)],
            pltpu.SemaphoreType.DMA,
        ],
    )
    def scatter_kernel(idx_hbm, w_hbm, p0_hbm, p1_hbm, acc, zbuf, xbufs,
                       tbufs, wbufs, slds, sscs, szb):
        cid = lax.axis_index("c")
        sid = lax.axis_index("s")
        wid = sid * _NC + cid
        m0 = wid * _NCHUNK

        def _start_load(c):
            b = c % 4
            m = m0 + c
            a, q = m // 32, m % 32
            ds_list = [pltpu.async_copy(
                w_hbm.at[pl.ds(m * _CHUNK, _CHUNK)], wbufs[b], slds[b])]
            for i in range(8):
                ds_list.append(pltpu.async_copy(
                    idx_hbm.at[pl.ds((8 * a + i) * _B + _STRIP * q, _STRIP)],
                    xbufs[b].at[pl.ds(i * _STRIP, _STRIP)], slds[b]))
            return ds_list

        lds = [_start_load(0), _start_load(1), None, None]

        def _zero_body(i, _):
            zbuf[pl.ds(i * 16, 16)] = jnp.zeros((16,), jnp.float32)
            return 0

        lax.fori_loop(0, _ZBUF // 16, _zero_body, 0)
        acc_base = sid * _ACC_PER_TILE
        zds = [
            pltpu.async_copy(
                zbuf, acc.at[pl.ds(acc_base + k * _ZBUF, _ZBUF)], szb)
            for k in range(_ACC_PER_TILE // _ZBUF)
        ]
        for d in zds:
            d.wait()
        plsc.subcore_barrier()

        scats = [None, None, None, None]
        for c in range(_NCHUNK):
            b = c % 4
            for d in lds[b]:
                d.wait()
            xb, tb = xbufs[b], tbufs[b]

            @plsc.parallel_loop(0, _CHUNK // 16, 1, unroll=8)
            def _reorder(t):
                src = ((t // 8) % 8) * _STRIP + (t // 64) * 128 + (t % 8) * 16
                tb[pl.ds(t * 16, 16)] = xb[pl.ds(src, 16)]

            scats[b] = pltpu.async_copy(
                wbufs[b], acc.at[tbufs[b]], sscs[b], add=True)
            if c >= 2:
                scats[(c - 2) % 4].wait()
                scats[(c - 2) % 4] = None
            if c + 2 < _NCHUNK:
                lds[(c + 2) % 4] = _start_load(c + 2)
        for s in scats:
            if s is not None:
                s.wait()

        plsc.subcore_barrier()
        sl = pl.ds(acc_base, _ACC_PER_TILE)

        @pl.when(cid == 0)
        def _():
            pltpu.sync_copy(acc.at[sl], p0_hbm.at[sl])

        @pl.when(cid == 1)
        def _():
            pltpu.sync_copy(acc.at[sl], p1_hbm.at[sl])

    return scatter_kernel(idx_hbm_arr, w_hbm_arr)


def _sc_combine(p0_arr, p1_arr):
    mesh = plsc.VectorSubcoreMesh(core_axis_name="c", subcore_axis_name="s")

    @functools.partial(
        pl.kernel,
        mesh=mesh,
        out_type=jax.ShapeDtypeStruct((_M,), jnp.float32),
        scratch_types=[
            pltpu.VMEM((_CW,), jnp.float32),
            pltpu.VMEM((_CW,), jnp.float32),
            pltpu.VMEM((_TAIL,), jnp.float32),
            pltpu.VMEM((_TAIL,), jnp.float32),
            pltpu.SemaphoreType.DMA,
        ],
    )
    def combine_kernel(p0_hbm, p1_hbm, out_hbm, b0, b1, t0, t1, sem):
        cid = lax.axis_index("c")
        sid = lax.axis_index("s")
        wid = sid * _NC + cid
        off = wid * _CW
        d0 = pltpu.async_copy(p0_hbm.at[pl.ds(off, _CW)], b0, sem)
        d1 = pltpu.async_copy(p1_hbm.at[pl.ds(off, _CW)], b1, sem)
        d0.wait()
        d1.wait()

        @plsc.parallel_loop(0, _CW // 16, 1, unroll=8)
        def _add(i):
            plsc.addupdate(b0.at[pl.ds(i * 16, 16)], b1[pl.ds(i * 16, 16)])

        pltpu.sync_copy(b0, out_hbm.at[pl.ds(off, _CW)])

        @pl.when(wid == 0)
        def _():
            base = _NW * _CW
            s = pl.ds(base, _TAIL)
            pltpu.sync_copy(p0_hbm.at[s], t0)
            pltpu.sync_copy(p1_hbm.at[s], t1)

            @plsc.parallel_loop(0, _TAIL // 16, 1, unroll=4)
            def _add_tail(k):
                plsc.addupdate(
                    t0.at[pl.ds(k * 16, 16)], t1[pl.ds(k * 16, 16)])

            pltpu.sync_copy(t0, out_hbm.at[s])

    return combine_kernel(p0_arr, p1_arr)


def kernel(x, w):
    idx_flat = x.transpose((1, 2, 0)).reshape(_N)
    w_flat = (w.transpose((1, 0))
              .reshape(_L // 8, 8, _B // 128, 128)
              .transpose((0, 2, 1, 3))
              .reshape(_N))
    p0, p1 = _sc_scatter_partials(idx_flat, w_flat)
    return _sc_combine(p0, p1)
